# baseline (device time: 63411 ns/iter reference)
import jax
import jax.numpy as jnp
from jax import lax
from jax.experimental import pallas as pl
from jax.experimental.pallas import tpu as pltpu

N_DEV = 4


def kernel(x, W1, W2):
    m_per, d = x.shape
    _, f_per = W1.shape

    def body(x_ref, w1_ref, w2_ref, out_ref,
             comm_ref, send_buf, recv_buf,
             ag_send_sems, ag_recv_sems, rs_send_sems, rs_recv_sems):
        my = lax.axis_index("i")
        left = (my - 1) % N_DEV
        right = (my + 1) % N_DEV

        barrier_sem = pltpu.get_barrier_semaphore()
        for nbr in (left, right):
            pl.semaphore_signal(
                barrier_sem, inc=1,
                device_id=(nbr,), device_id_type=pl.DeviceIdType.MESH,
            )
        pl.semaphore_wait(barrier_sem, 2)

        comm_ref[0] = x_ref[...].astype(jnp.bfloat16)
        w1b = w1_ref[...].astype(jnp.bfloat16)
        w2b = w2_ref[...].astype(jnp.bfloat16)

        def chunk_partial(xc):
            h = jnp.dot(xc, w1b, preferred_element_type=jnp.float32)
            h = (h * jax.nn.sigmoid(h)).astype(jnp.bfloat16)
            return jnp.dot(h, w2b, preferred_element_type=jnp.float32)

        for h in range(N_DEV - 1):
            rdma = pltpu.make_async_remote_copy(
                src_ref=comm_ref.at[h],
                dst_ref=comm_ref.at[h + 1],
                send_sem=ag_send_sems.at[h],
                recv_sem=ag_recv_sems.at[h],
                device_id=(right,),
                device_id_type=pl.DeviceIdType.MESH,
            )
            rdma.start()
            rdma.wait()

        p = [chunk_partial(comm_ref[s]) for s in range(N_DEV)]

        send_buf[0] = p[1].astype(jnp.bfloat16)
        acc = None
        for s in range(N_DEV - 1):
            rdma = pltpu.make_async_remote_copy(
                src_ref=send_buf.at[s],
                dst_ref=recv_buf.at[s],
                send_sem=rs_send_sems.at[s],
                recv_sem=rs_recv_sems.at[s],
                device_id=(right,),
                device_id_type=pl.DeviceIdType.MESH,
            )
            rdma.start()
            rdma.wait()
            acc = recv_buf[s].astype(jnp.float32) + p[(s + 2) % N_DEV]
            if s < N_DEV - 2:
                send_buf[s + 1] = acc.astype(jnp.bfloat16)

        out_ref[...] = acc

    return pl.pallas_call(
        body,
        out_shape=jax.ShapeDtypeStruct((m_per, d), jnp.float32),
        in_specs=[
            pl.BlockSpec(memory_space=pltpu.VMEM),
            pl.BlockSpec(memory_space=pltpu.VMEM),
            pl.BlockSpec(memory_space=pltpu.VMEM),
        ],
        out_specs=pl.BlockSpec(memory_space=pltpu.VMEM),
        scratch_shapes=[
            pltpu.VMEM((N_DEV, m_per, d), jnp.bfloat16),
            pltpu.VMEM((N_DEV - 1, m_per, d), jnp.bfloat16),
            pltpu.VMEM((N_DEV - 1, m_per, d), jnp.bfloat16),
            pltpu.SemaphoreType.DMA((N_DEV - 1,)),
            pltpu.SemaphoreType.DMA((N_DEV - 1,)),
            pltpu.SemaphoreType.DMA((N_DEV - 1,)),
            pltpu.SemaphoreType.DMA((N_DEV - 1,)),
        ],
        compiler_params=pltpu.CompilerParams(collective_id=0),
    )(x, W1, W2)


# device time: 48848 ns/iter; 1.2981x vs baseline; 1.2981x over previous
import jax
import jax.numpy as jnp
from jax import lax
from jax.experimental import pallas as pl
from jax.experimental.pallas import tpu as pltpu

N_DEV = 4


def kernel(x, W1, W2):
    m_per, d = x.shape
    _, f_per = W1.shape

    def body(x_ref, w1_ref, w2_ref, out_ref,
             comm_ref, send_buf, recv_buf,
             ag_send_sems, ag_recv_sems, rs_send_sems, rs_recv_sems):
        my = lax.axis_index("i")
        left = (my - 1) % N_DEV
        right = (my + 1) % N_DEV

        barrier_sem = pltpu.get_barrier_semaphore()
        for nbr in (left, right):
            pl.semaphore_signal(
                barrier_sem, inc=1,
                device_id=(nbr,), device_id_type=pl.DeviceIdType.MESH,
            )
        pl.semaphore_wait(barrier_sem, 2)

        comm_ref[0] = x_ref[...].astype(jnp.bfloat16)

        def ag_rdma(h):
            return pltpu.make_async_remote_copy(
                src_ref=comm_ref.at[h],
                dst_ref=comm_ref.at[h + 1],
                send_sem=ag_send_sems.at[h],
                recv_sem=ag_recv_sems.at[h],
                device_id=(right,),
                device_id_type=pl.DeviceIdType.MESH,
            )

        def rs_rdma(s):
            return pltpu.make_async_remote_copy(
                src_ref=send_buf.at[s],
                dst_ref=recv_buf.at[s],
                send_sem=rs_send_sems.at[s],
                recv_sem=rs_recv_sems.at[s],
                device_id=(right,),
                device_id_type=pl.DeviceIdType.MESH,
            )

        ag0 = ag_rdma(0)
        ag0.start()

        w1b = w1_ref[...].astype(jnp.bfloat16)
        w2b = w2_ref[...].astype(jnp.bfloat16)

        def chunk_partial(xc):
            h = jnp.dot(xc, w1b, preferred_element_type=jnp.float32)
            h = (h * jax.nn.sigmoid(h)).astype(jnp.bfloat16)
            return jnp.dot(h, w2b, preferred_element_type=jnp.float32)

        p0 = chunk_partial(comm_ref[0])

        ag0.wait_recv()
        ag1 = ag_rdma(1)
        ag1.start()

        p1 = chunk_partial(comm_ref[1])
        send_buf[0] = p1.astype(jnp.bfloat16)
        rs0 = rs_rdma(0)
        rs0.start()

        ag1.wait_recv()
        ag2 = ag_rdma(2)
        ag2.start()

        p2 = chunk_partial(comm_ref[2])

        rs0.wait_recv()
        a0 = recv_buf[0].astype(jnp.float32) + p2
        send_buf[1] = a0.astype(jnp.bfloat16)
        rs1 = rs_rdma(1)
        rs1.start()

        ag2.wait_recv()
        p3 = chunk_partial(comm_ref[3])

        rs1.wait_recv()
        a1 = recv_buf[1].astype(jnp.float32) + p3
        send_buf[2] = a1.astype(jnp.bfloat16)
        rs2 = rs_rdma(2)
        rs2.start()

        rs2.wait_recv()
        out_ref[...] = recv_buf[2].astype(jnp.float32) + p0

        for r in (ag0, ag1, ag2, rs0, rs1, rs2):
            r.wait_send()

    return pl.pallas_call(
        body,
        out_shape=jax.ShapeDtypeStruct((m_per, d), jnp.float32),
        in_specs=[
            pl.BlockSpec(memory_space=pltpu.VMEM),
            pl.BlockSpec(memory_space=pltpu.VMEM),
            pl.BlockSpec(memory_space=pltpu.VMEM),
        ],
        out_specs=pl.BlockSpec(memory_space=pltpu.VMEM),
        scratch_shapes=[
            pltpu.VMEM((N_DEV, m_per, d), jnp.bfloat16),
            pltpu.VMEM((N_DEV - 1, m_per, d), jnp.bfloat16),
            pltpu.VMEM((N_DEV - 1, m_per, d), jnp.bfloat16),
            pltpu.SemaphoreType.DMA((N_DEV - 1,)),
            pltpu.SemaphoreType.DMA((N_DEV - 1,)),
            pltpu.SemaphoreType.DMA((N_DEV - 1,)),
            pltpu.SemaphoreType.DMA((N_DEV - 1,)),
        ],
        compiler_params=pltpu.CompilerParams(collective_id=0),
    )(x, W1, W2)


# device time: 35728 ns/iter; 1.7748x vs baseline; 1.3672x over previous
import jax
import jax.numpy as jnp
from jax import lax
from jax.experimental import pallas as pl
from jax.experimental.pallas import tpu as pltpu

N_DEV = 4

XL, XR, XF = 0, 1, 2
QL, QR, QF = 0, 1, 2
AGR1, AGR2, AGL, FWD1, FWD2, RSL, RSR1, RSR2, RSF1, RSF2 = range(10)


def kernel(x, W1, W2):
    m_per, d = x.shape
    half = m_per // 2

    def body(x_ref, w1_ref, w2_ref, out_ref,
             xmine, xbuf, psend, pbuf, send_sems, recv_sems):
        my = lax.axis_index("i")
        left = (my - 1) % N_DEV
        right = (my + 1) % N_DEV
        opp = (my + 2) % N_DEV

        barrier_sem = pltpu.get_barrier_semaphore()
        for nbr in (left, right, opp):
            pl.semaphore_signal(
                barrier_sem, inc=1,
                device_id=(nbr,), device_id_type=pl.DeviceIdType.MESH,
            )
        pl.semaphore_wait(barrier_sem, 3)

        def copy(src, dst, slot, target):
            return pltpu.make_async_remote_copy(
                src_ref=src, dst_ref=dst,
                send_sem=send_sems.at[slot], recv_sem=recv_sems.at[slot],
                device_id=(target,), device_id_type=pl.DeviceIdType.MESH,
            )

        xmine[0] = x_ref[:half].astype(jnp.bfloat16)
        xmine[1] = x_ref[half:].astype(jnp.bfloat16)

        ag_r0 = copy(xmine.at[0], xbuf.at[XL, 0], AGR1, right)
        ag_r1 = copy(xmine.at[1], xbuf.at[XL, 1], AGR2, right)
        ag_l = copy(xmine, xbuf.at[XR], AGL, left)
        ag_r0.start()
        ag_r1.start()
        ag_l.start()

        w1b = w1_ref[...].astype(jnp.bfloat16)
        w2b = w2_ref[...].astype(jnp.bfloat16)

        def partial_h(xc):
            h = jnp.dot(xc, w1b, preferred_element_type=jnp.float32)
            h = (h * jax.nn.sigmoid(h)).astype(jnp.bfloat16)
            return jnp.dot(h, w2b, preferred_element_type=jnp.float32)

        p_own0 = partial_h(xmine[0])
        p_own1 = partial_h(xmine[1])

        copy(xmine.at[0], xbuf.at[XL, 0], AGR1, left).wait_recv()
        fwd0 = copy(xbuf.at[XL, 0], xbuf.at[XF, 0], FWD1, right)
        fwd0.start()
        copy(xmine.at[1], xbuf.at[XL, 1], AGR2, left).wait_recv()
        fwd1 = copy(xbuf.at[XL, 1], xbuf.at[XF, 1], FWD2, right)
        fwd1.start()

        psend[QL, 0] = partial_h(xbuf[XL, 0]).astype(jnp.bfloat16)
        psend[QL, 1] = partial_h(xbuf[XL, 1]).astype(jnp.bfloat16)
        rs_l = copy(psend.at[QL], pbuf.at[QR], RSL, left)
        rs_l.start()

        copy(xmine.at[0], xbuf.at[XF, 0], FWD1, left).wait_recv()
        psend[QF, 0] = partial_h(xbuf[XF, 0]).astype(jnp.bfloat16)
        rs_f0 = copy(psend.at[QF, 0], pbuf.at[QF, 0], RSF1, opp)
        rs_f0.start()

        copy(xmine, xbuf.at[XR], AGL, right).wait_recv()
        psend[QR, 0] = partial_h(xbuf[XR, 0]).astype(jnp.bfloat16)
        rs_r0 = copy(psend.at[QR, 0], pbuf.at[QL, 0], RSR1, right)
        rs_r0.start()

        copy(xmine.at[1], xbuf.at[XF, 1], FWD2, left).wait_recv()
        psend[QF, 1] = partial_h(xbuf[XF, 1]).astype(jnp.bfloat16)
        rs_f1 = copy(psend.at[QF, 1], pbuf.at[QF, 1], RSF2, opp)
        rs_f1.start()

        psend[QR, 1] = partial_h(xbuf[XR, 1]).astype(jnp.bfloat16)
        rs_r1 = copy(psend.at[QR, 1], pbuf.at[QL, 1], RSR2, right)
        rs_r1.start()

        copy(xmine, pbuf.at[QR], RSL, right).wait_recv()
        copy(xmine.at[0], pbuf.at[QL, 0], RSR1, left).wait_recv()
        copy(xmine.at[1], pbuf.at[QL, 1], RSR2, left).wait_recv()
        copy(xmine.at[0], pbuf.at[QF, 0], RSF1, opp).wait_recv()
        copy(xmine.at[1], pbuf.at[QF, 1], RSF2, opp).wait_recv()

        out_ref[:half] = (
            p_own0
            + pbuf[QL, 0].astype(jnp.float32)
            + pbuf[QR, 0].astype(jnp.float32)
            + pbuf[QF, 0].astype(jnp.float32)
        )
        out_ref[half:] = (
            p_own1
            + pbuf[QL, 1].astype(jnp.float32)
            + pbuf[QR, 1].astype(jnp.float32)
            + pbuf[QF, 1].astype(jnp.float32)
        )

        for r in (ag_r0, ag_r1, ag_l, fwd0, fwd1, rs_l, rs_r0, rs_r1,
                  rs_f0, rs_f1):
            r.wait_send()

    return pl.pallas_call(
        body,
        out_shape=jax.ShapeDtypeStruct((m_per, d), jnp.float32),
        in_specs=[
            pl.BlockSpec(memory_space=pltpu.VMEM),
            pl.BlockSpec(memory_space=pltpu.VMEM),
            pl.BlockSpec(memory_space=pltpu.VMEM),
        ],
        out_specs=pl.BlockSpec(memory_space=pltpu.VMEM),
        scratch_shapes=[
            pltpu.VMEM((2, half, d), jnp.bfloat16),
            pltpu.VMEM((3, 2, half, d), jnp.bfloat16),
            pltpu.VMEM((3, 2, half, d), jnp.bfloat16),
            pltpu.VMEM((3, 2, half, d), jnp.bfloat16),
            pltpu.SemaphoreType.DMA((10,)),
            pltpu.SemaphoreType.DMA((10,)),
        ],
        compiler_params=pltpu.CompilerParams(collective_id=0),
    )(x, W1, W2)


# device time: 35727 ns/iter; 1.7749x vs baseline; 1.0000x over previous
import jax
import jax.numpy as jnp
from jax import lax
from jax.experimental import pallas as pl
from jax.experimental.pallas import tpu as pltpu

N_DEV = 4

XL, XR, XF = 0, 1, 2
QL, QR, QF = 0, 1, 2
(AGR1, AGR2, AGL, FWD1, FWD2, RSL1, RSL2, RSR1, RSR2, RSF1,
 RSF2) = range(11)


def kernel(x, W1, W2):
    m_per, d = x.shape
    half = m_per // 2

    def body(x_ref, w1_ref, w2_ref, out_ref,
             xmine, xbuf, psend, pbuf, send_sems, recv_sems):
        my = lax.axis_index("i")
        left = (my - 1) % N_DEV
        right = (my + 1) % N_DEV
        opp = (my + 2) % N_DEV

        barrier_sem = pltpu.get_barrier_semaphore()
        for nbr in (left, right, opp):
            pl.semaphore_signal(
                barrier_sem, inc=1,
                device_id=(nbr,), device_id_type=pl.DeviceIdType.MESH,
            )
        pl.semaphore_wait(barrier_sem, 3)

        def copy(src, dst, slot, target):
            return pltpu.make_async_remote_copy(
                src_ref=src, dst_ref=dst,
                send_sem=send_sems.at[slot], recv_sem=recv_sems.at[slot],
                device_id=(target,), device_id_type=pl.DeviceIdType.MESH,
            )

        xmine[0] = x_ref[:half].astype(jnp.bfloat16)
        xmine[1] = x_ref[half:].astype(jnp.bfloat16)

        ag_r0 = copy(xmine.at[0], xbuf.at[XL, 0], AGR1, right)
        ag_r1 = copy(xmine.at[1], xbuf.at[XL, 1], AGR2, right)
        ag_l = copy(xmine, xbuf.at[XR], AGL, left)
        ag_r0.start()
        ag_r1.start()
        ag_l.start()

        w1b = w1_ref[...].astype(jnp.bfloat16)
        w2b = w2_ref[...].astype(jnp.bfloat16)

        def partial_h(xc):
            h = jnp.dot(xc, w1b, preferred_element_type=jnp.float32)
            h = (h * jax.nn.sigmoid(h)).astype(jnp.bfloat16)
            return jnp.dot(h, w2b, preferred_element_type=jnp.float32)

        p_own0 = partial_h(xmine[0])

        copy(xmine.at[0], xbuf.at[XL, 0], AGR1, left).wait_recv()
        fwd0 = copy(xbuf.at[XL, 0], xbuf.at[XF, 0], FWD1, right)
        fwd0.start()
        psend[QL, 0] = partial_h(xbuf[XL, 0]).astype(jnp.bfloat16)
        rs_l0 = copy(psend.at[QL, 0], pbuf.at[QR, 0], RSL1, left)
        rs_l0.start()

        copy(xmine.at[1], xbuf.at[XL, 1], AGR2, left).wait_recv()
        fwd1 = copy(xbuf.at[XL, 1], xbuf.at[XF, 1], FWD2, right)
        fwd1.start()
        psend[QL, 1] = partial_h(xbuf[XL, 1]).astype(jnp.bfloat16)
        rs_l1 = copy(psend.at[QL, 1], pbuf.at[QR, 1], RSL2, left)
        rs_l1.start()

        copy(xmine.at[0], xbuf.at[XF, 0], FWD1, left).wait_recv()
        psend[QF, 0] = partial_h(xbuf[XF, 0]).astype(jnp.bfloat16)
        rs_f0 = copy(psend.at[QF, 0], pbuf.at[QF, 0], RSF1, opp)
        rs_f0.start()

        copy(xmine, xbuf.at[XR], AGL, right).wait_recv()
        psend[QR, 0] = partial_h(xbuf[XR, 0]).astype(jnp.bfloat16)
        rs_r0 = copy(psend.at[QR, 0], pbuf.at[QL, 0], RSR1, right)
        rs_r0.start()

        copy(xmine.at[1], xbuf.at[XF, 1], FWD2, left).wait_recv()
        psend[QF, 1] = partial_h(xbuf[XF, 1]).astype(jnp.bfloat16)
        rs_f1 = copy(psend.at[QF, 1], pbuf.at[QF, 1], RSF2, opp)
        rs_f1.start()

        psend[QR, 1] = partial_h(xbuf[XR, 1]).astype(jnp.bfloat16)
        rs_r1 = copy(psend.at[QR, 1], pbuf.at[QL, 1], RSR2, right)
        rs_r1.start()

        p_own1 = partial_h(xmine[1])

        copy(xmine.at[0], pbuf.at[QR, 0], RSL1, right).wait_recv()
        copy(xmine.at[1], pbuf.at[QR, 1], RSL2, right).wait_recv()
        copy(xmine.at[0], pbuf.at[QL, 0], RSR1, left).wait_recv()
        copy(xmine.at[1], pbuf.at[QL, 1], RSR2, left).wait_recv()
        copy(xmine.at[0], pbuf.at[QF, 0], RSF1, opp).wait_recv()
        copy(xmine.at[1], pbuf.at[QF, 1], RSF2, opp).wait_recv()

        out_ref[:half] = (
            p_own0
            + pbuf[QL, 0].astype(jnp.float32)
            + pbuf[QR, 0].astype(jnp.float32)
            + pbuf[QF, 0].astype(jnp.float32)
        )
        out_ref[half:] = (
            p_own1
            + pbuf[QL, 1].astype(jnp.float32)
            + pbuf[QR, 1].astype(jnp.float32)
            + pbuf[QF, 1].astype(jnp.float32)
        )

        for r in (ag_r0, ag_r1, ag_l, fwd0, fwd1, rs_l0, rs_l1, rs_r0,
                  rs_r1, rs_f0, rs_f1):
            r.wait_send()

    return pl.pallas_call(
        body,
        out_shape=jax.ShapeDtypeStruct((m_per, d), jnp.float32),
        in_specs=[
            pl.BlockSpec(memory_space=pltpu.VMEM),
            pl.BlockSpec(memory_space=pltpu.VMEM),
            pl.BlockSpec(memory_space=pltpu.VMEM),
        ],
        out_specs=pl.BlockSpec(memory_space=pltpu.VMEM),
        scratch_shapes=[
            pltpu.VMEM((2, half, d), jnp.bfloat16),
            pltpu.VMEM((3, 2, half, d), jnp.bfloat16),
            pltpu.VMEM((3, 2, half, d), jnp.bfloat16),
            pltpu.VMEM((3, 2, half, d), jnp.bfloat16),
            pltpu.SemaphoreType.DMA((11,)),
            pltpu.SemaphoreType.DMA((11,)),
        ],
        compiler_params=pltpu.CompilerParams(collective_id=0),
    )(x, W1, W2)
